# Initial kernel scaffold; baseline (speedup 1.0000x reference)
import jax
import jax.numpy as jnp
from jax.experimental import pallas as pl


def kernel(x, edge_idx, W1, b1, W2, b2, W3, b3):
    raise NotImplementedError("write your pallas kernel here")


# speedup vs baseline: 18.4494x; 18.4494x over previous
import functools

import jax
import jax.numpy as jnp
from jax import lax
from jax.experimental import pallas as pl
from jax.experimental.pallas import tpu as pltpu, tpu_sc as plsc

N = 10000
E = 320000
IN_DIM = 128
HID = 64
OUT = 3

NC = 2
NS = 16
CH = 128
NCHUNK = 80
EPAD = NC * NS * NCHUNK * CH
R = 10112
ROWS_PER_TILE = R // NS

_mesh = plsc.VectorSubcoreMesh(
    core_axis_name="c", subcore_axis_name="s", num_cores=NC, num_subcores=NS
)
_sc_params = pltpu.CompilerParams(use_tc_tiling_on_sc=False)


def _make_deg_kernel():
  @functools.partial(
      pl.kernel,
      out_type=jax.ShapeDtypeStruct((NC, R, 8), jnp.float32),
      mesh=_mesh,
      compiler_params=_sc_params,
      scratch_types=[
          pltpu.VMEM((NCHUNK, CH), jnp.int32),
          pltpu.VMEM((CH, 8), jnp.float32),
          pltpu.VMEM_SHARED((R, 8), jnp.float32),
      ],
  )
  def deg_kernel(dst_hbm, ones_hbm, zero_hbm, out_hbm, dstb, onesb, accum):
    c = lax.axis_index("c")
    s = lax.axis_index("s")
    row0 = s * ROWS_PER_TILE
    pltpu.sync_copy(
        zero_hbm.at[pl.ds(row0, ROWS_PER_TILE)],
        accum.at[pl.ds(row0, ROWS_PER_TILE)],
    )
    pltpu.sync_copy(dst_hbm.at[c, s], dstb)
    pltpu.sync_copy(ones_hbm, onesb)
    plsc.subcore_barrier()

    @pl.loop(0, NCHUNK)
    def _(j):
      pltpu.sync_copy(onesb, accum.at[dstb.at[j]], add=True)

    plsc.subcore_barrier()
    pltpu.sync_copy(
        accum.at[pl.ds(row0, ROWS_PER_TILE)],
        out_hbm.at[c, pl.ds(row0, ROWS_PER_TILE)],
    )

  return deg_kernel


def _make_prop_kernel(d):
  @functools.partial(
      pl.kernel,
      out_type=jax.ShapeDtypeStruct((NC, R, d), jnp.float32),
      mesh=_mesh,
      compiler_params=_sc_params,
      scratch_types=[
          pltpu.VMEM((NCHUNK, CH), jnp.int32),
          pltpu.VMEM((NCHUNK, CH), jnp.int32),
          pltpu.VMEM((CH, d), jnp.float32),
          pltpu.VMEM((CH, d), jnp.float32),
          pltpu.SemaphoreType.DMA,
          pltpu.SemaphoreType.DMA,
          pltpu.VMEM_SHARED((R, d), jnp.float32),
      ],
  )
  def prop_kernel(y_hbm, src_hbm, dst_hbm, zero_hbm, out_hbm,
                  srcb, dstb, rows0, rows1, sem0, sem1, accum):
    c = lax.axis_index("c")
    s = lax.axis_index("s")
    row0 = s * ROWS_PER_TILE
    pltpu.sync_copy(
        zero_hbm.at[pl.ds(row0, ROWS_PER_TILE)],
        accum.at[pl.ds(row0, ROWS_PER_TILE)],
    )
    pltpu.sync_copy(src_hbm.at[c, s], srcb)
    pltpu.sync_copy(dst_hbm.at[c, s], dstb)
    plsc.subcore_barrier()

    cp0 = pltpu.async_copy(y_hbm.at[srcb.at[0]], rows0, sem0)

    @pl.loop(0, NCHUNK - 2, step=2)
    def _(j):
      cp1 = pltpu.async_copy(y_hbm.at[srcb.at[j + 1]], rows1, sem1)
      pltpu.make_async_copy(y_hbm.at[srcb.at[j]], rows0, sem0).wait()
      pltpu.sync_copy(rows0, accum.at[dstb.at[j]], add=True)
      pltpu.async_copy(y_hbm.at[srcb.at[j + 2]], rows0, sem0)
      pltpu.make_async_copy(y_hbm.at[srcb.at[j + 1]], rows1, sem1).wait()
      pltpu.sync_copy(rows1, accum.at[dstb.at[j + 1]], add=True)

    cpl = pltpu.async_copy(y_hbm.at[srcb.at[NCHUNK - 1]], rows1, sem1)
    pltpu.make_async_copy(y_hbm.at[srcb.at[NCHUNK - 2]], rows0, sem0).wait()
    pltpu.sync_copy(rows0, accum.at[dstb.at[NCHUNK - 2]], add=True)
    cpl.wait()
    pltpu.sync_copy(rows1, accum.at[dstb.at[NCHUNK - 1]], add=True)

    plsc.subcore_barrier()
    pltpu.sync_copy(
        accum.at[pl.ds(row0, ROWS_PER_TILE)],
        out_hbm.at[c, pl.ds(row0, ROWS_PER_TILE)],
    )

  return prop_kernel


_deg_kernel = _make_deg_kernel()
_prop64 = _make_prop_kernel(HID)
_prop8 = _make_prop_kernel(8)


def _tc_first(x, w1, degp):
  def body(x_ref, w1_ref, degp_ref, y_ref, dinv_ref):
    deg = degp_ref[0, :, 0:1] + degp_ref[1, :, 0:1] + 1.0
    dinv = lax.rsqrt(deg)
    xw = jnp.dot(x_ref[...], w1_ref[...], preferred_element_type=jnp.float32)
    y_ref[...] = xw * dinv
    dinv_ref[...] = dinv

  return pl.pallas_call(
      body,
      out_shape=(
          jax.ShapeDtypeStruct((R, HID), jnp.float32),
          jax.ShapeDtypeStruct((R, 1), jnp.float32),
      ),
  )(x, w1, degp)


def _tc_mid(s, y, dinv, b, w, d_out):
  def body(s_ref, y_ref, dinv_ref, b_ref, w_ref, yo_ref):
    h = dinv_ref[...] * (s_ref[0] + s_ref[1] + y_ref[...]) + b_ref[...]
    h = jnp.maximum(h, 0.0)
    yo_ref[...] = (
        jnp.dot(h, w_ref[...], preferred_element_type=jnp.float32)
        * dinv_ref[...]
    )

  return pl.pallas_call(
      body,
      out_shape=jax.ShapeDtypeStruct((R, d_out), jnp.float32),
  )(s, y, dinv, b, w)


def _tc_last(s, y, dinv, b):
  def body(s_ref, y_ref, dinv_ref, b_ref, o_ref):
    o_ref[...] = dinv_ref[...] * (s_ref[0] + s_ref[1] + y_ref[...]) + b_ref[...]

  return pl.pallas_call(
      body,
      out_shape=jax.ShapeDtypeStruct((R, 8), jnp.float32),
  )(s, y, dinv, b)


@jax.jit
def kernel(x, edge_idx, W1, b1, W2, b2, W3, b3):
  src = edge_idx[0].astype(jnp.int32)
  dst = edge_idx[1].astype(jnp.int32)
  pad = EPAD - E
  padv = jnp.full((pad,), N, dtype=jnp.int32)
  src4 = jnp.concatenate([src, padv]).reshape(NC, NS, NCHUNK, CH)
  dst4 = jnp.concatenate([dst, padv]).reshape(NC, NS, NCHUNK, CH)

  xp = jnp.zeros((R, IN_DIM), jnp.float32).at[:N].set(x)
  w3p = jnp.zeros((HID, 8), jnp.float32).at[:, :OUT].set(W3)
  b3p = jnp.zeros((1, 8), jnp.float32).at[0, :OUT].set(b3)
  ones8 = jnp.ones((CH, 8), jnp.float32)
  z64 = jnp.zeros((R, HID), jnp.float32)
  z8 = jnp.zeros((R, 8), jnp.float32)

  degp = _deg_kernel(dst4, ones8, z8)
  y1, dinv = _tc_first(xp, W1, degp)
  s1 = _prop64(y1, src4, dst4, z64)
  y2 = _tc_mid(s1, y1, dinv, b1.reshape(1, HID), W2, HID)
  s2 = _prop64(y2, src4, dst4, z64)
  y3 = _tc_mid(s2, y2, dinv, b2.reshape(1, HID), w3p, 8)
  s3 = _prop8(y3, src4, dst4, z8)
  outp = _tc_last(s3, y3, dinv, b3p)
  return outp[:N, :OUT]


# speedup vs baseline: 18.6539x; 1.0111x over previous
import functools

import jax
import jax.numpy as jnp
from jax import lax
from jax.experimental import pallas as pl
from jax.experimental.pallas import tpu as pltpu, tpu_sc as plsc

N = 10000
E = 320000
IN_DIM = 128
HID = 64
OUT = 3

NC = 2
NS = 16
CH = 128
NCHUNK = 80
NBUF = 8
EPAD = NC * NS * NCHUNK * CH
R = 10112
ROWS_PER_TILE = R // NS

_mesh = plsc.VectorSubcoreMesh(
    core_axis_name="c", subcore_axis_name="s", num_cores=NC, num_subcores=NS
)
_sc_params = pltpu.CompilerParams(use_tc_tiling_on_sc=False)


def _make_deg_kernel():
  @functools.partial(
      pl.kernel,
      out_type=jax.ShapeDtypeStruct((NC, R, 8), jnp.float32),
      mesh=_mesh,
      compiler_params=_sc_params,
      scratch_types=[
          pltpu.VMEM((NCHUNK, CH), jnp.int32),
          pltpu.VMEM((CH, 8), jnp.float32),
          pltpu.SemaphoreType.DMA,
          pltpu.VMEM_SHARED((R, 8), jnp.float32),
      ],
  )
  def deg_kernel(dst_hbm, ones_hbm, zero_hbm, out_hbm, dstb, onesb, sem, accum):
    c = lax.axis_index("c")
    s = lax.axis_index("s")
    row0 = s * ROWS_PER_TILE
    pltpu.sync_copy(
        zero_hbm.at[pl.ds(row0, ROWS_PER_TILE)],
        accum.at[pl.ds(row0, ROWS_PER_TILE)],
    )
    pltpu.sync_copy(dst_hbm.at[c, s], dstb)
    pltpu.sync_copy(ones_hbm, onesb)
    plsc.subcore_barrier()

    @pl.loop(0, NCHUNK, step=16)
    def _(j):
      for u in range(16):
        pltpu.async_copy(onesb, accum.at[dstb.at[j + u]], sem, add=True)
      for u in range(16):
        pltpu.make_async_copy(onesb, accum.at[dstb.at[j + u]], sem).wait()

    plsc.subcore_barrier()
    pltpu.sync_copy(
        accum.at[pl.ds(row0, ROWS_PER_TILE)],
        out_hbm.at[c, pl.ds(row0, ROWS_PER_TILE)],
    )

  return deg_kernel


def _make_prop_kernel(d):
  @functools.partial(
      pl.kernel,
      out_type=jax.ShapeDtypeStruct((NC, R, d), jnp.float32),
      mesh=_mesh,
      compiler_params=_sc_params,
      scratch_types=[
          pltpu.VMEM((NCHUNK, CH), jnp.int32),
          pltpu.VMEM((NCHUNK, CH), jnp.int32),
          [pltpu.VMEM((CH, d), jnp.float32)] * NBUF,
          [pltpu.SemaphoreType.DMA] * NBUF,
          [pltpu.SemaphoreType.DMA] * NBUF,
          pltpu.VMEM_SHARED((R, d), jnp.float32),
      ],
  )
  def prop_kernel(y_hbm, src_hbm, dst_hbm, zero_hbm, out_hbm,
                  srcb, dstb, rows, gsem, ssem, accum):
    c = lax.axis_index("c")
    s = lax.axis_index("s")
    row0 = s * ROWS_PER_TILE
    pltpu.sync_copy(
        zero_hbm.at[pl.ds(row0, ROWS_PER_TILE)],
        accum.at[pl.ds(row0, ROWS_PER_TILE)],
    )
    pltpu.sync_copy(src_hbm.at[c, s], srcb)
    pltpu.sync_copy(dst_hbm.at[c, s], dstb)
    plsc.subcore_barrier()

    def gather(j, b):
      pltpu.async_copy(y_hbm.at[srcb.at[j]], rows[b], gsem[b])

    def scatter(j, b):
      pltpu.async_copy(rows[b], accum.at[dstb.at[j]], ssem[b], add=True)

    def gather_wait(j, b):
      pltpu.make_async_copy(y_hbm.at[srcb.at[j]], rows[b], gsem[b]).wait()

    def scatter_wait(j, b):
      pltpu.make_async_copy(rows[b], accum.at[dstb.at[j]], ssem[b]).wait()

    for b in range(NBUF):
      gather(b, b)

    @pl.loop(0, NCHUNK - NBUF, step=NBUF)
    def _(j):
      for b in range(NBUF):
        gather_wait(j + b, b)
        scatter(j + b, b)
      for b in range(NBUF):
        scatter_wait(j + b, b)
        gather(j + b + NBUF, b)

    last = NCHUNK - NBUF
    for b in range(NBUF):
      gather_wait(last + b, b)
      scatter(last + b, b)
    for b in range(NBUF):
      scatter_wait(last + b, b)

    plsc.subcore_barrier()
    pltpu.sync_copy(
        accum.at[pl.ds(row0, ROWS_PER_TILE)],
        out_hbm.at[c, pl.ds(row0, ROWS_PER_TILE)],
    )

  return prop_kernel


_deg_kernel = _make_deg_kernel()
_prop64 = _make_prop_kernel(HID)
_prop8 = _make_prop_kernel(8)


def _tc_first(x, w1, degp):
  def body(x_ref, w1_ref, degp_ref, y_ref, dinv_ref):
    deg = degp_ref[0, :, 0:1] + degp_ref[1, :, 0:1] + 1.0
    dinv = lax.rsqrt(deg)
    xw = jnp.dot(x_ref[...], w1_ref[...], preferred_element_type=jnp.float32)
    y_ref[...] = xw * dinv
    dinv_ref[...] = dinv

  return pl.pallas_call(
      body,
      out_shape=(
          jax.ShapeDtypeStruct((R, HID), jnp.float32),
          jax.ShapeDtypeStruct((R, 1), jnp.float32),
      ),
  )(x, w1, degp)


def _tc_mid(s, y, dinv, b, w, d_out):
  def body(s_ref, y_ref, dinv_ref, b_ref, w_ref, yo_ref):
    h = dinv_ref[...] * (s_ref[0] + s_ref[1] + y_ref[...]) + b_ref[...]
    h = jnp.maximum(h, 0.0)
    yo_ref[...] = (
        jnp.dot(h, w_ref[...], preferred_element_type=jnp.float32)
        * dinv_ref[...]
    )

  return pl.pallas_call(
      body,
      out_shape=jax.ShapeDtypeStruct((R, d_out), jnp.float32),
  )(s, y, dinv, b, w)


def _tc_last(s, y, dinv, b):
  def body(s_ref, y_ref, dinv_ref, b_ref, o_ref):
    o_ref[...] = dinv_ref[...] * (s_ref[0] + s_ref[1] + y_ref[...]) + b_ref[...]

  return pl.pallas_call(
      body,
      out_shape=jax.ShapeDtypeStruct((R, 8), jnp.float32),
  )(s, y, dinv, b)


@jax.jit
def kernel(x, edge_idx, W1, b1, W2, b2, W3, b3):
  src = edge_idx[0].astype(jnp.int32)
  dst = edge_idx[1].astype(jnp.int32)
  pad = EPAD - E
  padv = jnp.full((pad,), N, dtype=jnp.int32)
  padd = N + (jnp.arange(pad, dtype=jnp.int32) % (R - N))
  src4 = jnp.concatenate([src, padv]).reshape(NC, NS, NCHUNK, CH)
  dst4 = jnp.concatenate([dst, padd]).reshape(NC, NS, NCHUNK, CH)

  xp = jnp.zeros((R, IN_DIM), jnp.float32).at[:N].set(x)
  w3p = jnp.zeros((HID, 8), jnp.float32).at[:, :OUT].set(W3)
  b3p = jnp.zeros((1, 8), jnp.float32).at[0, :OUT].set(b3)
  ones8 = jnp.ones((CH, 8), jnp.float32)
  z64 = jnp.zeros((R, HID), jnp.float32)
  z8 = jnp.zeros((R, 8), jnp.float32)

  degp = _deg_kernel(dst4, ones8, z8)
  y1, dinv = _tc_first(xp, W1, degp)
  s1 = _prop64(y1, src4, dst4, z64)
  y2 = _tc_mid(s1, y1, dinv, b1.reshape(1, HID), W2, HID)
  s2 = _prop64(y2, src4, dst4, z64)
  y3 = _tc_mid(s2, y2, dinv, b2.reshape(1, HID), w3p, 8)
  s3 = _prop8(y3, src4, dst4, z8)
  outp = _tc_last(s3, y3, dinv, b3p)
  return outp[:N, :OUT]


# speedup vs baseline: 31.9493x; 1.7127x over previous
import functools

import jax
import jax.numpy as jnp
from jax import lax
from jax.experimental import pallas as pl
from jax.experimental.pallas import tpu as pltpu, tpu_sc as plsc

N = 10000
E = 320000
IN_DIM = 128
HID = 64
OUT = 3

NC = 2
NS = 16
CH = 128
NCHUNK = 80
NBUF = 8
EPAD = NC * NS * NCHUNK * CH
R = 10112
ROWS_PER_TILE = R // NS

_mesh = plsc.VectorSubcoreMesh(
    core_axis_name="c", subcore_axis_name="s", num_cores=NC, num_subcores=NS
)
_sc_params = pltpu.CompilerParams(use_tc_tiling_on_sc=False)


def _make_deg_kernel():
  @functools.partial(
      pl.kernel,
      out_type=jax.ShapeDtypeStruct((NC, R, 8), jnp.float32),
      mesh=_mesh,
      compiler_params=_sc_params,
      scratch_types=[
          pltpu.VMEM((NCHUNK, CH), jnp.int32),
          pltpu.VMEM((CH, 8), jnp.float32),
          pltpu.SemaphoreType.DMA,
          pltpu.VMEM_SHARED((R, 8), jnp.float32),
      ],
  )
  def deg_kernel(dst_hbm, ones_hbm, zero_hbm, out_hbm, dstb, onesb, sem, accum):
    c = lax.axis_index("c")
    s = lax.axis_index("s")
    row0 = s * ROWS_PER_TILE
    pltpu.sync_copy(
        zero_hbm.at[pl.ds(row0, ROWS_PER_TILE)],
        accum.at[pl.ds(row0, ROWS_PER_TILE)],
    )
    pltpu.sync_copy(dst_hbm.at[c, s], dstb)
    pltpu.sync_copy(ones_hbm, onesb)
    plsc.subcore_barrier()

    @pl.loop(0, NCHUNK, step=16)
    def _(j):
      for u in range(16):
        pltpu.async_copy(onesb, accum.at[dstb.at[j + u]], sem, add=True)
      for u in range(16):
        pltpu.make_async_copy(onesb, accum.at[dstb.at[j + u]], sem).wait()

    plsc.subcore_barrier()
    pltpu.sync_copy(
        accum.at[pl.ds(row0, ROWS_PER_TILE)],
        out_hbm.at[c, pl.ds(row0, ROWS_PER_TILE)],
    )

  return deg_kernel


def _make_prop_kernel(p_passes, w):
  @functools.partial(
      pl.kernel,
      out_type=jax.ShapeDtypeStruct((NC, p_passes, R, w), jnp.float32),
      mesh=_mesh,
      compiler_params=_sc_params,
      scratch_types=[
          pltpu.VMEM((NCHUNK, CH), jnp.int32),
          pltpu.VMEM((NCHUNK, CH), jnp.int32),
          [pltpu.VMEM((CH, w), jnp.float32)] * NBUF,
          [pltpu.SemaphoreType.DMA] * NBUF,
          [pltpu.SemaphoreType.DMA] * NBUF,
          pltpu.VMEM_SHARED((R, w), jnp.float32),
          pltpu.VMEM_SHARED((R, w), jnp.float32),
      ],
  )
  def prop_kernel(y_hbm, src_hbm, dst_hbm, zero_hbm, out_hbm,
                  srcb, dstb, rows, gsem, ssem, accum, y_sp):
    c = lax.axis_index("c")
    s = lax.axis_index("s")
    row0 = s * ROWS_PER_TILE
    pltpu.sync_copy(src_hbm.at[c, s], srcb)
    pltpu.sync_copy(dst_hbm.at[c, s], dstb)

    def gather(j, b):
      pltpu.async_copy(y_sp.at[srcb.at[j]], rows[b], gsem[b])

    def scatter(j, b):
      pltpu.async_copy(rows[b], accum.at[dstb.at[j]], ssem[b], add=True)

    def gather_wait(j, b):
      pltpu.make_async_copy(y_sp.at[srcb.at[j]], rows[b], gsem[b]).wait()

    def scatter_wait(j, b):
      pltpu.make_async_copy(rows[b], accum.at[dstb.at[j]], ssem[b]).wait()

    for p in range(p_passes):
      pltpu.sync_copy(
          y_hbm.at[p, pl.ds(row0, ROWS_PER_TILE)],
          y_sp.at[pl.ds(row0, ROWS_PER_TILE)],
      )
      pltpu.sync_copy(
          zero_hbm.at[pl.ds(row0, ROWS_PER_TILE)],
          accum.at[pl.ds(row0, ROWS_PER_TILE)],
      )
      plsc.subcore_barrier()

      for b in range(NBUF):
        gather(b, b)

      @pl.loop(0, NCHUNK - NBUF, step=NBUF)
      def _(j):
        for b in range(NBUF):
          gather_wait(j + b, b)
          scatter(j + b, b)
        for b in range(NBUF):
          scatter_wait(j + b, b)
          gather(j + b + NBUF, b)

      last = NCHUNK - NBUF
      for b in range(NBUF):
        gather_wait(last + b, b)
        scatter(last + b, b)
      for b in range(NBUF):
        scatter_wait(last + b, b)

      plsc.subcore_barrier()
      pltpu.sync_copy(
          accum.at[pl.ds(row0, ROWS_PER_TILE)],
          out_hbm.at[c, p, pl.ds(row0, ROWS_PER_TILE)],
      )

  return prop_kernel


_deg_kernel = _make_deg_kernel()
_prop64 = _make_prop_kernel(2, HID // 2)
_prop8 = _make_prop_kernel(1, 8)


def _tc_first(x, w1, degp):
  def body(x_ref, w1_ref, degp_ref, y_ref, dinv_ref):
    deg = degp_ref[0, :, 0:1] + degp_ref[1, :, 0:1] + 1.0
    dinv = lax.rsqrt(deg)
    xw = jnp.dot(x_ref[...], w1_ref[...], preferred_element_type=jnp.float32)
    y = xw * dinv
    y_ref[0] = y[:, : HID // 2]
    y_ref[1] = y[:, HID // 2 :]
    dinv_ref[...] = dinv

  return pl.pallas_call(
      body,
      out_shape=(
          jax.ShapeDtypeStruct((2, R, HID // 2), jnp.float32),
          jax.ShapeDtypeStruct((R, 1), jnp.float32),
      ),
  )(x, w1, degp)


def _tc_mid(s, y, dinv, b, w, p_out, w_out):
  def body(s_ref, y_ref, dinv_ref, b_ref, w_ref, yo_ref):
    conv = jnp.concatenate(
        [
            s_ref[0, 0] + s_ref[1, 0] + y_ref[0],
            s_ref[0, 1] + s_ref[1, 1] + y_ref[1],
        ],
        axis=1,
    )
    h = jnp.maximum(dinv_ref[...] * conv + b_ref[...], 0.0)
    yo = (
        jnp.dot(h, w_ref[...], preferred_element_type=jnp.float32)
        * dinv_ref[...]
    )
    if p_out == 1:
      yo_ref[0] = yo
    else:
      yo_ref[0] = yo[:, :w_out]
      yo_ref[1] = yo[:, w_out:]

  return pl.pallas_call(
      body,
      out_shape=jax.ShapeDtypeStruct((p_out, R, w_out), jnp.float32),
  )(s, y, dinv, b, w)


def _tc_last(s, y, dinv, b):
  def body(s_ref, y_ref, dinv_ref, b_ref, o_ref):
    o_ref[...] = (
        dinv_ref[...] * (s_ref[0, 0] + s_ref[1, 0] + y_ref[0]) + b_ref[...]
    )

  return pl.pallas_call(
      body,
      out_shape=jax.ShapeDtypeStruct((R, 8), jnp.float32),
  )(s, y, dinv, b)


@jax.jit
def kernel(x, edge_idx, W1, b1, W2, b2, W3, b3):
  src = edge_idx[0].astype(jnp.int32)
  dst = edge_idx[1].astype(jnp.int32)
  pad = EPAD - E
  padv = jnp.full((pad,), N, dtype=jnp.int32)
  padd = N + (jnp.arange(pad, dtype=jnp.int32) % (R - N))
  src4 = jnp.concatenate([src, padv]).reshape(NC, NS, NCHUNK, CH)
  dst4 = jnp.concatenate([dst, padd]).reshape(NC, NS, NCHUNK, CH)

  xp = jnp.zeros((R, IN_DIM), jnp.float32).at[:N].set(x)
  w3p = jnp.zeros((HID, 8), jnp.float32).at[:, :OUT].set(W3)
  b3p = jnp.zeros((1, 8), jnp.float32).at[0, :OUT].set(b3)
  ones8 = jnp.ones((CH, 8), jnp.float32)
  z32 = jnp.zeros((R, HID // 2), jnp.float32)
  z8 = jnp.zeros((R, 8), jnp.float32)

  degp = _deg_kernel(dst4, ones8, z8)
  y1, dinv = _tc_first(xp, W1, degp)
  s1 = _prop64(y1, src4, dst4, z32)
  y2 = _tc_mid(s1, y1, dinv, b1.reshape(1, HID), W2, 2, HID // 2)
  s2 = _prop64(y2, src4, dst4, z32)
  y3 = _tc_mid(s2, y2, dinv, b2.reshape(1, HID), w3p, 1, 8)
  s3 = _prop8(y3, src4, dst4, z8)
  outp = _tc_last(s3, y3, dinv, b3p)
  return outp[:N, :OUT]


# speedup vs baseline: 33.4131x; 1.0458x over previous
import functools

import jax
import jax.numpy as jnp
from jax import lax
from jax.experimental import pallas as pl
from jax.experimental.pallas import tpu as pltpu, tpu_sc as plsc

N = 10000
E = 320000
IN_DIM = 128
HID = 64
OUT = 3

NC = 2
NS = 16
CH = 125
NCHUNK = 80
NBUF = 8
ROWS_PER_TILE = N // NS

_mesh = plsc.VectorSubcoreMesh(
    core_axis_name="c", subcore_axis_name="s", num_cores=NC, num_subcores=NS
)
_sc_params = pltpu.CompilerParams(use_tc_tiling_on_sc=False)


def _make_deg_kernel():
  @functools.partial(
      pl.kernel,
      out_type=jax.ShapeDtypeStruct((NC, N, 8), jnp.float32),
      mesh=_mesh,
      compiler_params=_sc_params,
      scratch_types=[
          pltpu.VMEM((NCHUNK, CH), jnp.int32),
          pltpu.VMEM((CH, 8), jnp.float32),
          pltpu.SemaphoreType.DMA,
          pltpu.VMEM_SHARED((N, 8), jnp.float32),
      ],
  )
  def deg_kernel(dst_hbm, ones_hbm, zero_hbm, out_hbm, dstb, onesb, sem, accum):
    c = lax.axis_index("c")
    s = lax.axis_index("s")
    row0 = s * ROWS_PER_TILE
    pltpu.sync_copy(
        zero_hbm.at[pl.ds(row0, ROWS_PER_TILE)],
        accum.at[pl.ds(row0, ROWS_PER_TILE)],
    )
    pltpu.sync_copy(dst_hbm.at[c, s], dstb)
    pltpu.sync_copy(ones_hbm, onesb)
    plsc.subcore_barrier()

    @pl.loop(0, NCHUNK, step=16)
    def _(j):
      for u in range(16):
        pltpu.async_copy(onesb, accum.at[dstb.at[j + u]], sem, add=True)
      for u in range(16):
        pltpu.make_async_copy(onesb, accum.at[dstb.at[j + u]], sem).wait()

    plsc.subcore_barrier()
    pltpu.sync_copy(
        accum.at[pl.ds(row0, ROWS_PER_TILE)],
        out_hbm.at[c, pl.ds(row0, ROWS_PER_TILE)],
    )

  return deg_kernel


def _make_prop_kernel(p_passes, w):
  @functools.partial(
      pl.kernel,
      out_type=jax.ShapeDtypeStruct((NC, p_passes, N, w), jnp.float32),
      mesh=_mesh,
      compiler_params=_sc_params,
      scratch_types=[
          pltpu.VMEM((NCHUNK, CH), jnp.int32),
          pltpu.VMEM((NCHUNK, CH), jnp.int32),
          [pltpu.VMEM((CH, w), jnp.float32)] * NBUF,
          [pltpu.SemaphoreType.DMA] * NBUF,
          [pltpu.SemaphoreType.DMA] * NBUF,
          pltpu.VMEM_SHARED((N, w), jnp.float32),
          pltpu.VMEM_SHARED((N, w), jnp.float32),
      ],
  )
  def prop_kernel(y_hbm, src_hbm, dst_hbm, zero_hbm, out_hbm,
                  srcb, dstb, rows, gsem, ssem, accum, y_sp):
    c = lax.axis_index("c")
    s = lax.axis_index("s")
    row0 = s * ROWS_PER_TILE
    pltpu.sync_copy(src_hbm.at[c, s], srcb)
    pltpu.sync_copy(dst_hbm.at[c, s], dstb)

    def gather(j, b):
      pltpu.async_copy(y_sp.at[srcb.at[j]], rows[b], gsem[b])

    def scatter(j, b):
      pltpu.async_copy(rows[b], accum.at[dstb.at[j]], ssem[b], add=True)

    def gather_wait(j, b):
      pltpu.make_async_copy(y_sp.at[srcb.at[j]], rows[b], gsem[b]).wait()

    def scatter_wait(j, b):
      pltpu.make_async_copy(rows[b], accum.at[dstb.at[j]], ssem[b]).wait()

    for p in range(p_passes):
      pltpu.sync_copy(
          y_hbm.at[p, pl.ds(row0, ROWS_PER_TILE)],
          y_sp.at[pl.ds(row0, ROWS_PER_TILE)],
      )
      pltpu.sync_copy(
          zero_hbm.at[pl.ds(row0, ROWS_PER_TILE)],
          accum.at[pl.ds(row0, ROWS_PER_TILE)],
      )
      plsc.subcore_barrier()

      for b in range(NBUF):
        gather(b, b)

      @pl.loop(0, NCHUNK - NBUF, step=NBUF)
      def _(j):
        for b in range(NBUF):
          gather_wait(j + b, b)
          scatter(j + b, b)
        for b in range(NBUF):
          scatter_wait(j + b, b)
          gather(j + b + NBUF, b)

      last = NCHUNK - NBUF
      for b in range(NBUF):
        gather_wait(last + b, b)
        scatter(last + b, b)
      for b in range(NBUF):
        scatter_wait(last + b, b)

      plsc.subcore_barrier()
      pltpu.sync_copy(
          accum.at[pl.ds(row0, ROWS_PER_TILE)],
          out_hbm.at[c, p, pl.ds(row0, ROWS_PER_TILE)],
      )

  return prop_kernel


_deg_kernel = _make_deg_kernel()
_prop64 = _make_prop_kernel(2, HID // 2)
_prop8 = _make_prop_kernel(1, 8)


def _tc_matmul(x, w):
  def body(x_ref, w_ref, o_ref):
    o_ref[...] = jnp.dot(
        x_ref[...], w_ref[...], preferred_element_type=jnp.float32
    )

  return pl.pallas_call(
      body,
      out_shape=jax.ShapeDtypeStruct((N, w.shape[1]), jnp.float32),
  )(x, w)


def _tc_scale(xw, degp):
  def body(xw_ref, degp_ref, y_ref, dinv_ref):
    deg = degp_ref[0, :, 0:1] + degp_ref[1, :, 0:1] + 1.0
    dinv = lax.rsqrt(deg)
    y = xw_ref[...] * dinv
    y_ref[0] = y[:, : HID // 2]
    y_ref[1] = y[:, HID // 2 :]
    dinv_ref[...] = dinv

  return pl.pallas_call(
      body,
      out_shape=(
          jax.ShapeDtypeStruct((2, N, HID // 2), jnp.float32),
          jax.ShapeDtypeStruct((N, 1), jnp.float32),
      ),
  )(xw, degp)


def _tc_mid(s, y, dinv, b, w, p_out, w_out):
  def body(s_ref, y_ref, dinv_ref, b_ref, w_ref, yo_ref):
    conv = jnp.concatenate(
        [
            s_ref[0, 0] + s_ref[1, 0] + y_ref[0],
            s_ref[0, 1] + s_ref[1, 1] + y_ref[1],
        ],
        axis=1,
    )
    h = jnp.maximum(dinv_ref[...] * conv + b_ref[...], 0.0)
    yo = (
        jnp.dot(h, w_ref[...], preferred_element_type=jnp.float32)
        * dinv_ref[...]
    )
    if p_out == 1:
      yo_ref[0] = yo
    else:
      yo_ref[0] = yo[:, :w_out]
      yo_ref[1] = yo[:, w_out:]

  return pl.pallas_call(
      body,
      out_shape=jax.ShapeDtypeStruct((p_out, N, w_out), jnp.float32),
  )(s, y, dinv, b, w)


def _tc_last(s, y, dinv, b):
  def body(s_ref, y_ref, dinv_ref, b_ref, o_ref):
    o_ref[...] = (
        dinv_ref[...] * (s_ref[0, 0] + s_ref[1, 0] + y_ref[0]) + b_ref[...]
    )

  return pl.pallas_call(
      body,
      out_shape=jax.ShapeDtypeStruct((N, 8), jnp.float32),
  )(s, y, dinv, b)


@jax.jit
def kernel(x, edge_idx, W1, b1, W2, b2, W3, b3):
  src4 = edge_idx[0].astype(jnp.int32).reshape(NC, NS, NCHUNK, CH)
  dst4 = edge_idx[1].astype(jnp.int32).reshape(NC, NS, NCHUNK, CH)

  w3p = jnp.zeros((HID, 8), jnp.float32).at[:, :OUT].set(W3)
  b3p = jnp.zeros((1, 8), jnp.float32).at[0, :OUT].set(b3)
  ones8 = jnp.ones((CH, 8), jnp.float32)
  z32 = jnp.zeros((N, HID // 2), jnp.float32)
  z8 = jnp.zeros((N, 8), jnp.float32)

  degp = _deg_kernel(dst4, ones8, z8)
  xw1 = _tc_matmul(x, W1)
  y1, dinv = _tc_scale(xw1, degp)
  s1 = _prop64(y1, src4, dst4, z32)
  y2 = _tc_mid(s1, y1, dinv, b1.reshape(1, HID), W2, 2, HID // 2)
  s2 = _prop64(y2, src4, dst4, z32)
  y3 = _tc_mid(s2, y2, dinv, b2.reshape(1, HID), w3p, 1, 8)
  s3 = _prop8(y3, src4, dst4, z8)
  outp = _tc_last(s3, y3, dinv, b3p)
  return outp[:, :OUT]
